# baseline (speedup 1.0000x reference)
import jax
import jax.numpy as jnp
from jax.experimental import pallas as pl
from jax.experimental.pallas import tpu as pltpu

N = 16384
C = 1000
BLK = 2048
NB = N // BLK
K = int(N * 0.7)
MININT = -2147483648


def _ce_kernel(tgt_ref, xt_ref, loss_ref):
    x = xt_ref[...]
    t = tgt_ref[...]
    m = jnp.max(x, axis=0, keepdims=True)
    s = jnp.sum(jnp.exp(x - m), axis=0, keepdims=True)
    logz = m + jnp.log(s)
    rows = jax.lax.broadcasted_iota(jnp.int32, (C, BLK), 0)
    picked = jnp.sum(jnp.where(rows == t, x, 0.0), axis=0, keepdims=True)
    loss_ref[...] = logz - picked


def _thresh_kernel(loss_ref, out_ref):
    loss = loss_ref[...]
    kb = jax.lax.bitcast_convert_type(loss, jnp.int32)
    keys = kb ^ (jax.lax.shift_right_arithmetic(kb, 31) & jnp.int32(0x7FFFFFFF))

    def body(j, t_u):
        bit = jax.lax.shift_left(jnp.int32(1), 31 - j)
        cand = t_u | bit
        cnt = jnp.sum((keys >= (cand ^ jnp.int32(MININT))).astype(jnp.int32))
        return jnp.where(cnt >= K, cand, t_u)

    t_u = jax.lax.fori_loop(0, 32, body, jnp.int32(0))
    thr = t_u ^ jnp.int32(MININT)
    mask = keys >= thr
    s_h = jnp.sum(jnp.where(mask, loss, 0.0))
    c_h = jnp.sum(mask.astype(jnp.float32))
    out_ref[...] = (s_h / c_h).reshape(1, 1)


@jax.jit
def kernel(predictions, targets):
    xt = predictions.T
    tgt = targets.astype(jnp.int32).reshape(1, N)
    loss = pl.pallas_call(
        _ce_kernel,
        grid=(NB,),
        in_specs=[
            pl.BlockSpec((1, BLK), lambda i: (0, i)),
            pl.BlockSpec((C, BLK), lambda i: (0, i)),
        ],
        out_specs=pl.BlockSpec((1, BLK), lambda i: (0, i)),
        out_shape=jax.ShapeDtypeStruct((1, N), jnp.float32),
        compiler_params=pltpu.CompilerParams(
            dimension_semantics=("parallel",),
        ),
    )(tgt, xt)
    out = pl.pallas_call(
        _thresh_kernel,
        out_shape=jax.ShapeDtypeStruct((1, 1), jnp.float32),
    )(loss)
    return out[0, 0]


# speedup vs baseline: 1.2574x; 1.2574x over previous
import jax
import jax.numpy as jnp
from jax.experimental import pallas as pl
from jax.experimental.pallas import tpu as pltpu

N = 16384
C = 1000
BLK = 2048
NB = N // BLK
NCH = C // 8
K = int(N * 0.7)
MININT = -2147483648


def _ohem_kernel(tgt_ref, xt_ref, out_ref, loss_ref):
    i = pl.program_id(0)
    t = tgt_ref[...]
    tb = jnp.broadcast_to(t, (8, BLK))
    sub = jax.lax.broadcasted_iota(jnp.int32, (8, BLK), 0)

    m_acc = jnp.full((8, BLK), -jnp.inf, jnp.float32)
    p_acc = jnp.zeros((8, BLK), jnp.float32)
    for r in range(NCH):
        x = xt_ref[8 * r : 8 * r + 8, :]
        m_acc = jnp.maximum(m_acc, x)
        p_acc = jnp.where(tb == sub + (8 * r), x, p_acc)
    m = jnp.max(m_acc, axis=0, keepdims=True)
    picked = jnp.sum(p_acc, axis=0, keepdims=True)

    mb = jnp.broadcast_to(m, (8, BLK))
    s_acc = jnp.zeros((8, BLK), jnp.float32)
    for r in range(NCH):
        x = xt_ref[8 * r : 8 * r + 8, :]
        s_acc = s_acc + jnp.exp(x - mb)
    s = jnp.sum(s_acc, axis=0, keepdims=True)

    logz = m + jnp.log(s)
    loss_ref[i, :] = (logz - picked)[0, :]

    @pl.when(i == NB - 1)
    def _tail():
        loss = loss_ref[...]
        kb = jax.lax.bitcast_convert_type(loss, jnp.int32)
        keys = kb ^ (jax.lax.shift_right_arithmetic(kb, 31) & jnp.int32(0x7FFFFFFF))

        def body(j, t_u):
            bit = jax.lax.shift_left(jnp.int32(1), 31 - j)
            cand = t_u | bit
            cnt = jnp.sum((keys >= (cand ^ jnp.int32(MININT))).astype(jnp.int32))
            return jnp.where(cnt >= K, cand, t_u)

        t_u = jax.lax.fori_loop(0, 32, body, jnp.int32(0))
        thr = t_u ^ jnp.int32(MININT)
        mask = keys >= thr
        s_h = jnp.sum(jnp.where(mask, loss, 0.0))
        c_h = jnp.sum(mask.astype(jnp.float32))
        out_ref[...] = (s_h / c_h).reshape(1, 1)


@jax.jit
def kernel(predictions, targets):
    xt = predictions.T
    tgt = targets.astype(jnp.int32).reshape(1, N)
    out = pl.pallas_call(
        _ohem_kernel,
        grid=(NB,),
        in_specs=[
            pl.BlockSpec((1, BLK), lambda i: (0, i)),
            pl.BlockSpec((C, BLK), lambda i: (0, i)),
        ],
        out_specs=pl.BlockSpec((1, 1), lambda i: (0, 0)),
        out_shape=jax.ShapeDtypeStruct((1, 1), jnp.float32),
        scratch_shapes=[pltpu.VMEM((NB, BLK), jnp.float32)],
        compiler_params=pltpu.CompilerParams(
            dimension_semantics=("arbitrary",),
        ),
    )(tgt, xt)
    return out[0, 0]
